# Initial kernel scaffold; baseline (speedup 1.0000x reference)
import jax
import jax.numpy as jnp
from jax.experimental import pallas as pl


def kernel(x, ei, b, params):
    raise NotImplementedError("write your pallas kernel here")


# speedup vs baseline: 5.0116x; 5.0116x over previous
import functools

import jax
import jax.numpy as jnp
from jax import lax
from jax.experimental import pallas as pl
from jax.experimental.pallas import tpu as pltpu, tpu_sc as plsc

N = 10000
E = 320000
F = 128
G = 128

NW = 32
EPW = E // NW
C = 80
NCHUNK = EPW // C
NPAD = 10240
RPT = NPAD // 16


def _agg_body(with_counts, h_hbm, src_hbm, dst_hbm, *refs):
  if with_counts:
    agg_hbm, cnt_hbm = refs[0], refs[1]
    refs = refs[2:]
  else:
    agg_hbm = refs[0]
    refs = refs[1:]
  sidx_v, didx_v, rows_v, zbuf, ones_v, acc_sh, cnt_sh, sem = refs

  c = lax.axis_index("c")
  s = lax.axis_index("s")
  w = c * 16 + s

  zv = jnp.zeros((16,), jnp.float32)
  for r in range(32):
    for k in range(8):
      zbuf[r, pl.ds(k * 16, 16)] = zv

  for k in range(RPT // 32):
    pltpu.sync_copy(zbuf, acc_sh.at[pl.ds(s * RPT + k * 32, 32)])

  if with_counts:
    for k in range(C // 16):
      ones_v[pl.ds(k * 16, 16)] = jnp.ones((16,), jnp.float32)
    pltpu.sync_copy(zbuf.at[0].at[pl.ds(0, 64)],
                    cnt_sh.at[pl.ds(s * RPT, 64)])
    for k in range(1, RPT // 64):
      pltpu.sync_copy(zbuf.at[0].at[pl.ds(0, 64)],
                      cnt_sh.at[pl.ds(s * RPT + k * 64, 64)])

  plsc.subcore_barrier()

  @pl.loop(0, NCHUNK)
  def _(j):
    eb = pl.multiple_of(w * EPW + j * C, 8)
    pltpu.sync_copy(src_hbm.at[pl.ds(eb, C)], sidx_v)
    pltpu.sync_copy(dst_hbm.at[pl.ds(eb, C)], didx_v)
    pltpu.async_copy(h_hbm.at[sidx_v], rows_v, sem).wait()
    pltpu.sync_copy(rows_v, acc_sh.at[didx_v], add=True)
    if with_counts:
      pltpu.sync_copy(ones_v, cnt_sh.at[didx_v], add=True)

  plsc.subcore_barrier()

  pltpu.sync_copy(acc_sh.at[pl.ds(s * RPT, RPT)],
                  agg_hbm.at[c].at[pl.ds(s * RPT, RPT)])
  if with_counts:
    pltpu.sync_copy(cnt_sh.at[pl.ds(s * RPT, RPT)],
                    cnt_hbm.at[c].at[pl.ds(s * RPT, RPT)])


@functools.lru_cache(maxsize=None)
def _make_agg(with_counts):
  out_type = [jax.ShapeDtypeStruct((2, NPAD, F), jnp.float32)]
  if with_counts:
    out_type.append(jax.ShapeDtypeStruct((2, NPAD), jnp.float32))
  return pl.kernel(
      functools.partial(_agg_body, with_counts),
      out_type=out_type,
      mesh=plsc.VectorSubcoreMesh(core_axis_name="c", subcore_axis_name="s",
                                  num_cores=2, num_subcores=16),
      scratch_types=[
          pltpu.VMEM((C,), jnp.int32),
          pltpu.VMEM((C,), jnp.int32),
          pltpu.VMEM((C, F), jnp.float32),
          pltpu.VMEM((32, F), jnp.float32),
          pltpu.VMEM((C,), jnp.float32),
          pltpu.VMEM_SHARED((NPAD, F), jnp.float32),
          pltpu.VMEM_SHARED((NPAD,), jnp.float32),
          pltpu.SemaphoreType.DMA,
      ],
  )


def _dense_body(agg_ref, cnt_ref, h_ref, wl_ref, bl_ref, wr_ref, g_ref,
                bb_ref, out_ref):
  agg = agg_ref[0, :N, :] + agg_ref[1, :N, :]
  cnt = cnt_ref[0, :N] + cnt_ref[1, :N]
  inv = 1.0 / jnp.maximum(cnt, 1.0)
  mean = agg * inv[:, None]
  t = (jnp.dot(mean, wl_ref[...], preferred_element_type=jnp.float32)
       + jnp.dot(h_ref[...], wr_ref[...], preferred_element_type=jnp.float32)
       + bl_ref[...])
  mu = jnp.mean(t, axis=0)
  xc = t - mu
  var = jnp.mean(xc * xc, axis=0)
  y = xc * (g_ref[...] * jax.lax.rsqrt(var + 1e-5)) + bb_ref[...]
  out_ref[...] = jnp.maximum(y, 0.0)


_dense = pl.pallas_call(
    _dense_body,
    out_shape=jax.ShapeDtypeStruct((N, F), jnp.float32),
)


def _pool_head_body(h_ref, b_ref, w1_ref, b1_ref, w2_ref, b2_ref, out_ref):
  h = h_ref[...]
  bvec = b_ref[...]
  gids = lax.broadcasted_iota(jnp.int32, (G, N), 0)
  oht = (gids == bvec).astype(jnp.float32)
  s = jnp.dot(oht, h, preferred_element_type=jnp.float32)
  cg = jnp.sum(oht, axis=1)
  pooled = s * (1.0 / jnp.maximum(cg, 1.0))[:, None]
  z = jnp.maximum(
      jnp.dot(pooled, w1_ref[...], preferred_element_type=jnp.float32)
      + b1_ref[...], 0.0)
  out_ref[...] = (jnp.dot(z, w2_ref[...], preferred_element_type=jnp.float32)
                  + b2_ref[...])


_pool_head = pl.pallas_call(
    _pool_head_body,
    out_shape=jax.ShapeDtypeStruct((G, 128), jnp.float32),
)


def kernel(x, ei, b, params):
  src = ei[0]
  dst = ei[1]
  h = x
  cnt32 = None
  for i in range(3):
    if cnt32 is None:
      agg2, cnt32 = _make_agg(True)(h, src, dst)
    else:
      (agg2,) = _make_agg(False)(h, src, dst)
    cp = params["convs"][i]
    bn = params["bns"][i]
    h = _dense(agg2, cnt32, h, cp["Wl"].T, cp["bl"], cp["Wr"].T,
               bn["g"], bn["b"])
  hd = params["head"]
  return _pool_head(h, b.reshape(1, N), hd["W1"].T, hd["b1"], hd["W2"].T,
                    hd["b2"])
